# baseline (speedup 1.0000x reference)
import functools
import jax
import jax.numpy as jnp
from jax import lax
from jax.experimental import pallas as pl
from jax.experimental.pallas import tpu as pltpu
from jax.experimental.pallas import tpu_sc as plsc

B, C, H, W_SP, E, K = 4, 64, 128, 128, 64, 8
S = H * W_SP
T = B * S
TBLK = 2048
NS = S // TBLK

_SC_INFO = plsc.get_sparse_core_info()
NW = _SC_INFO.num_cores * _SC_INFO.num_subcores
L = _SC_INFO.num_lanes
TOK_W = T // NW
TCH = 128
NCH = TOK_W // TCH


def _router_body(x_ref, w_ref, b_ref, logits_ref, weights_ref, idx_ref,
                 idxt_ref):
    xb = x_ref[0]
    l = jnp.dot(w_ref[...], xb, preferred_element_type=jnp.float32)
    l = l + b_ref[...]
    logits_ref[0] = l

    m = jnp.max(l, axis=0, keepdims=True)
    e = jnp.exp(l - m)
    z = jnp.sum(e, axis=0, keepdims=True)
    p = e / z

    ii = lax.broadcasted_iota(jnp.int32, (C, TBLK), 0)
    vals, idxs = [], []
    cur = p
    for _ in range(K):
        mk = jnp.max(cur, axis=0, keepdims=True)
        sel = cur == mk
        ik = jnp.min(jnp.where(sel, ii, C), axis=0, keepdims=True)
        vals.append(mk)
        idxs.append(ik)
        cur = jnp.where(ii == ik, -1.0, cur)

    wv = jnp.concatenate(vals, axis=0)
    iv = jnp.concatenate(idxs, axis=0)
    weights_ref[0] = wv / jnp.sum(wv, axis=0, keepdims=True)
    idx_ref[0] = iv
    idxt_ref[...] = iv


def _tc_router(xr, W, br):
    return pl.pallas_call(
        _router_body,
        grid=(B, NS),
        in_specs=[
            pl.BlockSpec((1, C, TBLK), lambda bb, s: (bb, 0, s)),
            pl.BlockSpec((C, C), lambda bb, s: (0, 0)),
            pl.BlockSpec((C, 1), lambda bb, s: (0, 0)),
        ],
        out_specs=[
            pl.BlockSpec((1, C, TBLK), lambda bb, s: (bb, 0, s)),
            pl.BlockSpec((1, K, TBLK), lambda bb, s: (bb, 0, s)),
            pl.BlockSpec((1, K, TBLK), lambda bb, s: (bb, 0, s)),
            pl.BlockSpec((K, TBLK), lambda bb, s: (0, bb * NS + s)),
        ],
        out_shape=[
            jax.ShapeDtypeStruct((B, C, S), jnp.float32),
            jax.ShapeDtypeStruct((B, K, S), jnp.float32),
            jax.ShapeDtypeStruct((B, K, S), jnp.int32),
            jax.ShapeDtypeStruct((K, T), jnp.int32),
        ],
    )(xr, W, br)


def _mask_body(idxt_hbm, mask_hbm, idx_v, blk_v):
    wid = lax.axis_index("s") * _SC_INFO.num_cores + lax.axis_index("c")
    base = wid * TOK_W
    lanes = lax.iota(jnp.int32, L)
    ones = jnp.ones((L,), jnp.int32)
    zeros = jnp.zeros((L,), jnp.int32)

    def _zero(i, carry):
        row = i >> 3
        j = i & 7
        blk_v[row, pl.ds(j * L, L)] = zeros
        return carry

    lax.fori_loop(0, (E * K) * (TCH // L), _zero, 0)

    def _chunk(c, carry):
        t0 = base + c * TCH
        pltpu.sync_copy(idxt_hbm.at[:, pl.ds(t0, TCH)], idx_v)
        for k in range(K):
            for j in range(TCH // L):
                ev = idx_v[k, pl.ds(j * L, L)]
                row = ev * K + k
                col = lanes + (j * L)
                plsc.store_scatter(blk_v, [row, col], ones)
        pltpu.sync_copy(blk_v, mask_hbm.at[:, pl.ds(t0, TCH)])
        for k in range(K):
            for j in range(TCH // L):
                ev = idx_v[k, pl.ds(j * L, L)]
                row = ev * K + k
                col = lanes + (j * L)
                plsc.store_scatter(blk_v, [row, col], zeros)
        return carry

    lax.fori_loop(0, NCH, _chunk, 0)


def _sc_mask(idxt):
    mesh = plsc.VectorSubcoreMesh(core_axis_name="c", subcore_axis_name="s")
    f = functools.partial(
        pl.kernel,
        out_type=jax.ShapeDtypeStruct((E * K, T), jnp.int32),
        mesh=mesh,
        compiler_params=pltpu.CompilerParams(needs_layout_passes=False),
        scratch_types=[
            pltpu.VMEM((K, TCH), jnp.int32),
            pltpu.VMEM((E * K, TCH), jnp.int32),
        ],
    )(_mask_body)
    return f(idxt)


def kernel(x, W, b):
    xr = x.reshape(B, C, S)
    br = b.reshape(C, 1)
    logits, weights, idx, idxt = _tc_router(xr, W, br)
    mask = _sc_mask(idxt)
    return (
        logits.reshape(B, C, H, W_SP),
        weights.reshape(B, K, H, W_SP),
        idx.reshape(B, K, H, W_SP),
        mask.reshape(E, K, T),
    )


# speedup vs baseline: 2.2374x; 2.2374x over previous
import jax
import jax.numpy as jnp
from jax import lax
from jax.experimental import pallas as pl

B, C, H, W_SP, E, K = 4, 64, 128, 128, 64, 8
T = B * H * W_SP
HB = 16
TBLK = HB * W_SP
NH = H // HB


def _router_body(x_ref, w_ref, b_ref, logits_ref, weights_ref, idx_ref,
                 mask_ref):
    xb = x_ref[0].reshape(C, TBLK)
    l = jnp.dot(w_ref[...], xb, preferred_element_type=jnp.float32)
    l = l + b_ref[...]
    logits_ref[0] = l.reshape(C, HB, W_SP)

    m = jnp.max(l, axis=0, keepdims=True)
    e = jnp.exp(l - m)
    z = jnp.sum(e, axis=0, keepdims=True)
    p = e / z

    ii = lax.broadcasted_iota(jnp.int32, (C, TBLK), 0)
    vals, idxs = [], []
    cur = p
    for _ in range(K):
        mk = jnp.max(cur, axis=0, keepdims=True)
        sel = cur == mk
        ik = jnp.min(jnp.where(sel, ii, C), axis=0, keepdims=True)
        vals.append(mk)
        idxs.append(ik)
        cur = jnp.where(ii == ik, -1.0, cur)

    wv = jnp.concatenate(vals, axis=0)
    iv = jnp.concatenate(idxs, axis=0)
    wn = wv / jnp.sum(wv, axis=0, keepdims=True)
    weights_ref[0] = wn.reshape(K, HB, W_SP)
    idx_ref[0] = iv.reshape(K, HB, W_SP)

    ee = lax.broadcasted_iota(jnp.int32, (E, K, TBLK), 0)
    mask_ref[...] = (iv[None] == ee).astype(jnp.int32)


def kernel(x, W, b):
    br = b.reshape(C, 1)
    logits, weights, idx, mask = pl.pallas_call(
        _router_body,
        grid=(B, NH),
        in_specs=[
            pl.BlockSpec((1, C, HB, W_SP), lambda bb, h: (bb, 0, h, 0)),
            pl.BlockSpec((C, C), lambda bb, h: (0, 0)),
            pl.BlockSpec((C, 1), lambda bb, h: (0, 0)),
        ],
        out_specs=[
            pl.BlockSpec((1, C, HB, W_SP), lambda bb, h: (bb, 0, h, 0)),
            pl.BlockSpec((1, K, HB, W_SP), lambda bb, h: (bb, 0, h, 0)),
            pl.BlockSpec((1, K, HB, W_SP), lambda bb, h: (bb, 0, h, 0)),
            pl.BlockSpec((E, K, TBLK), lambda bb, h: (0, 0, bb * NH + h)),
        ],
        out_shape=[
            jax.ShapeDtypeStruct((B, C, H, W_SP), jnp.float32),
            jax.ShapeDtypeStruct((B, K, H, W_SP), jnp.float32),
            jax.ShapeDtypeStruct((B, K, H, W_SP), jnp.int32),
            jax.ShapeDtypeStruct((E, K, T), jnp.int32),
        ],
    )(x, W, br)
    return (logits, weights, idx, mask)
